# baseline (speedup 1.0000x reference)
import jax
import jax.numpy as jnp
from jax.experimental import pallas as pl
from jax.experimental.pallas import tpu as pltpu

_B, _N, _F, _MS, _PASSES = 2048, 32, 28, 128, 3
_NT = 4
_NA = _NT * _N
_BT = 128


def _dot(a, b):
    return jax.lax.dot_general(a, b, (((1,), (0,)), ((), ())),
                               preferred_element_type=jnp.float32)


def _bdot(a, b):
    return jax.lax.dot_general(a, b, (((2,), (1,)), ((0,), (0,))),
                               preferred_element_type=jnp.float32)


def _tile_kernel(ah_ref, al_ref, ab_ref, aa_ref, x_ref, wstk_ref, bin4_ref,
                 wself_ref, wmsg_ref, bmsg_ref, w1_ref, b1_ref, w2_ref,
                 b2_ref, out_ref):
    x2 = x_ref[...].reshape(_BT * _N, _NT * _F)
    h0 = jnp.tanh(_dot(x2, wstk_ref[...]) + bin4_ref[...])
    h = jnp.concatenate(
        [h0[:, t * _MS:(t + 1) * _MS].reshape(_BT, _N, _MS)
         for t in range(_NT)], axis=1)
    h = h.reshape(_BT * _NA, _MS)

    As = (ah_ref[...], al_ref[...], ab_ref[...], aa_ref[...])
    wself = wself_ref[...]
    wmsg = wmsg_ref[...]
    bmsg = bmsg_ref[...]
    for _ in range(_PASSES):
        h3 = h.reshape(_BT, _NA, _MS)
        m = jnp.concatenate(
            [_bdot(As[t], h3[:, t * _N:(t + 1) * _N, :])
             for t in range(_NT)], axis=1).reshape(_BT * _NA, _MS)
        h = jnp.tanh(_dot(h, wself) + _dot(m, wmsg) + bmsg)

    embs = jnp.sum(h.reshape(_BT, _NT, _N, _MS), axis=2)
    hidden = jnp.broadcast_to(b1_ref[...], (_BT, _NT * _MS))
    for t in range(_NT):
        hidden = hidden + _dot(embs[:, t, :], w1_ref[t * _MS:(t + 1) * _MS, :])
    hidden = jax.nn.relu(hidden)
    y = _dot(hidden, w2_ref[...]) + b2_ref[...]
    out_ref[...] = jnp.abs(y)


def kernel(halide_matrices, halide_features, ligand_matrices, ligand_features,
           base_matrices, base_features, additive_matrices, additive_features,
           W_in, b_in, W_self, W_msg, b_msg, W1, b1, W2, b2):
    mats = (halide_matrices, ligand_matrices, base_matrices, additive_matrices)
    feats = (halide_features, ligand_features, base_features, additive_features)
    X_cat = jnp.concatenate(feats, axis=2)
    W_stack = jnp.zeros((_NT * _F, _NT * _MS), jnp.float32)
    for t in range(_NT):
        W_stack = W_stack.at[t * _F:(t + 1) * _F,
                             t * _MS:(t + 1) * _MS].set(W_in)
    b_in4 = jnp.tile(b_in, _NT).reshape(1, _NT * _MS)

    grid = (_B // _BT,)

    def w_spec(shape):
        return pl.BlockSpec(shape, lambda i: tuple(0 for _ in shape))

    a_spec = pl.BlockSpec((_BT, _N, _N), lambda i: (i, 0, 0))

    out = pl.pallas_call(
        _tile_kernel,
        grid=grid,
        in_specs=[a_spec, a_spec, a_spec, a_spec,
                  pl.BlockSpec((_BT, _N, _NT * _F), lambda i: (i, 0, 0)),
                  w_spec((_NT * _F, _NT * _MS)), w_spec((1, _NT * _MS)),
                  w_spec((_MS, _MS)), w_spec((_MS, _MS)), w_spec((1, _MS)),
                  w_spec((_NT * _MS, _NT * _MS)), w_spec((1, _NT * _MS)),
                  w_spec((_NT * _MS, 1)), w_spec((1, 1))],
        out_specs=pl.BlockSpec((_BT, 1), lambda i: (i, 0)),
        out_shape=jax.ShapeDtypeStruct((_B, 1), jnp.float32),
        compiler_params=pltpu.CompilerParams(
            dimension_semantics=("arbitrary",)),
    )(*mats, X_cat,
      W_stack, b_in4, W_self, W_msg, b_msg.reshape(1, _MS),
      W1, b1.reshape(1, _NT * _MS), W2, b2.reshape(1, 1))
    return out.reshape(-1)
